# baseline (speedup 1.0000x reference)
import functools

import jax
import jax.numpy as jnp
from jax.experimental import pallas as pl

_H = 16
_LANE = 128


def _su(a):
    return jnp.concatenate([a[1:], jnp.zeros((1, a.shape[1]), a.dtype)], axis=0)


def _sd(a):
    return jnp.concatenate([jnp.zeros((1, a.shape[1]), a.dtype), a[:-1]], axis=0)


def _net_kernel(xp_ref, w0_ref,
                wa1_ref, wb1_ref, ws1_ref, g1_ref,
                wa2_ref, wb2_ref, ws2_ref, g2_ref,
                wa3_ref, wb3_ref, ws3_ref, g3_ref,
                wa4_ref, wb4_ref, ws4_ref, g4_ref,
                pool_ref, out_ref, *, n_nodes, block, tile):
    t = pl.program_id(0)
    X = xp_ref[pl.ds(t * block, tile), :]

    xn = _su(X)
    rel = xn - X
    d2 = jnp.sum(rel * rel, axis=1, keepdims=True)
    dr = jnp.sqrt(d2 + 1e-12)
    dl = _sd(dr)

    g = jax.lax.broadcasted_iota(jnp.int32, (tile, 1), 0) + (t * block - _H)
    mask_l = ((g >= 1) & (g <= n_nodes - 1)).astype(jnp.float32)
    mask_r = ((g >= 0) & (g <= n_nodes - 2)).astype(jnp.float32)
    inv_deg = 1.0 / jnp.maximum(mask_l + mask_r, 1.0)

    def coefs(width):
        s = width / 2.0

        def phi(dv, ring):
            z = (dv - ring) / s
            return jnp.exp(-0.5 * z * z)

        cl0 = phi(dl, 0.0) * mask_l * inv_deg
        cl1 = phi(dl, width) * mask_l * inv_deg
        cr0 = phi(dr, 0.0) * mask_r * inv_deg
        cr1 = phi(dr, width) * mask_r * inv_deg
        return cl0, cl1, cr0, cr1

    cf1 = coefs(1.0)
    cf10 = coefs(10.0)

    def pconv(xd, xu, cf, w_ref):
        cl0, cl1, cr0, cr1 = cf
        p = xd * cl0 + xu * cr0
        q = xd * cl1 + xu * cr1
        dn = (((1,), (1,)), ((), ()))
        return (jax.lax.dot_general(p, w_ref[0], dn,
                                    preferred_element_type=jnp.float32) +
                jax.lax.dot_general(q, w_ref[1], dn,
                                    preferred_element_type=jnp.float32))

    def gated(h, nff, g_ref):
        sig = jax.nn.sigmoid(h[:, nff:nff + 16])
        return h[:, :nff] * jnp.dot(sig, g_ref[:, :],
                                    preferred_element_type=jnp.float32)

    x = pconv(_sd(X), _su(X), cf1, w0_ref)

    for wa, wb, ws, gm, nff in (
        (wa1_ref, wb1_ref, ws1_ref, g1_ref, 80),
        (wa2_ref, wb2_ref, ws2_ref, g2_ref, 240),
        (wa3_ref, wb3_ref, ws3_ref, g3_ref, 480),
        (wa4_ref, wb4_ref, ws4_ref, g4_ref, 320),
    ):
        xd, xu = _sd(x), _su(x)
        h = gated(pconv(xd, xu, cf1, wa), nff, gm)
        h = gated(pconv(_sd(h), _su(h), cf1, wb), nff, gm)
        x = pconv(xd, xu, cf10, ws) + h

    out = jnp.sqrt(jnp.dot(x * x, pool_ref[:, :],
                           preferred_element_type=jnp.float32) + 1e-12)
    out_ref[:, :] = out[_H:_H + block, :]


def _pad_w(w, o_pad, i_pad):
    r, o, i = w.shape
    return jnp.pad(w, ((0, 0), (0, o_pad - o), (0, i_pad - i)))


def _gate_expand(nf, fd, nff):
    rows = jax.lax.broadcasted_iota(jnp.int32, (16, nff), 0)
    cols = jax.lax.broadcasted_iota(jnp.int32, (16, nff), 1)
    return (cols // fd == rows).astype(jnp.float32)


def _pool_mat(nf, fd, nff):
    rows = jax.lax.broadcasted_iota(jnp.int32, (nff, _LANE), 0)
    cols = jax.lax.broadcasted_iota(jnp.int32, (nff, _LANE), 1)
    return ((rows // fd == cols) & (cols < nf)).astype(jnp.float32)


@jax.jit
def kernel(input, W0, W1a, W1b, W1s, W2a, W2b, W2s, W3a, W3b, W3s, W4a,
           W4b, W4s):
    n = input.shape[0]
    block = 2000 if n >= 2000 else max(8, (n + 7) // 8 * 8)
    grid = -(-n // block)
    total = grid * block + 2 * _H
    tile = block + 2 * _H

    xp = jnp.zeros((total, _LANE), jnp.float32)
    xp = xp.at[_H:_H + n, :3].set(input)

    w0 = _pad_w(W0, 40, _LANE)
    wa1 = _pad_w(W1a, 96, 40)
    wb1 = _pad_w(W1b, 96, 80)
    ws1 = _pad_w(W1s, 80, 40)
    wa2 = _pad_w(W2a, 256, 80)
    wb2 = _pad_w(W2b, 256, 240)
    ws2 = _pad_w(W2s, 240, 80)
    wa3 = _pad_w(W3a, 496, 240)
    wb3 = _pad_w(W3b, 496, 480)
    ws3 = _pad_w(W3s, 480, 240)
    wa4 = _pad_w(W4a, 336, 480)
    wb4 = _pad_w(W4b, 336, 320)
    ws4 = _pad_w(W4s, 320, 480)
    g1 = _gate_expand(2, 40, 80)
    g2 = _gate_expand(6, 40, 240)
    g3 = _gate_expand(12, 40, 480)
    g4 = _gate_expand(8, 40, 320)
    pool = _pool_mat(8, 40, 320)

    def full(a):
        return pl.BlockSpec(a.shape, lambda t: (0,) * a.ndim)

    operands = (xp, w0, wa1, wb1, ws1, g1, wa2, wb2, ws2, g2,
                wa3, wb3, ws3, g3, wa4, wb4, ws4, g4, pool)
    out = pl.pallas_call(
        functools.partial(_net_kernel, n_nodes=n, block=block, tile=tile),
        grid=(grid,),
        in_specs=[full(a) for a in operands],
        out_specs=pl.BlockSpec((block, _LANE), lambda t: (t, 0)),
        out_shape=jax.ShapeDtypeStruct((grid * block, _LANE), jnp.float32),
    )(*operands)
    return out[:n, :8]


# speedup vs baseline: 1.0745x; 1.0745x over previous
import functools

import jax
import jax.numpy as jnp
from jax.experimental import pallas as pl

_H = 16
_LANE = 128


def _su(a):
    return jnp.concatenate([a[1:], jnp.zeros((1, a.shape[1]), a.dtype)], axis=0)


def _sd(a):
    return jnp.concatenate([jnp.zeros((1, a.shape[1]), a.dtype), a[:-1]], axis=0)


def _net_kernel(xp_ref, w0_ref,
                wa1_ref, wb1_ref, ws1_ref, g1_ref,
                wa2_ref, wb2_ref, ws2_ref, g2_ref,
                wa3_ref, wb3_ref, ws3_ref, g3_ref,
                wa4_ref, wb4_ref, ws4_ref, g4_ref,
                pool_ref, out_ref, *, n_nodes, block, tile):
    t = pl.program_id(0)
    X = xp_ref[pl.ds(t * block, tile), :]

    xn = _su(X)
    rel = xn - X
    d2 = jnp.sum(rel * rel, axis=1, keepdims=True)
    dr = jnp.sqrt(d2 + 1e-12)
    dl = _sd(dr)

    g = jax.lax.broadcasted_iota(jnp.int32, (tile, 1), 0) + (t * block - _H)
    mask_l = ((g >= 1) & (g <= n_nodes - 1)).astype(jnp.float32)
    mask_r = ((g >= 0) & (g <= n_nodes - 2)).astype(jnp.float32)
    inv_deg = 1.0 / jnp.maximum(mask_l + mask_r, 1.0)

    def coefs(width):
        s = width / 2.0

        def phi(dv, ring):
            z = (dv - ring) / s
            return jnp.exp(-0.5 * z * z)

        cl0 = phi(dl, 0.0) * mask_l * inv_deg
        cl1 = phi(dl, width) * mask_l * inv_deg
        cr0 = phi(dr, 0.0) * mask_r * inv_deg
        cr1 = phi(dr, width) * mask_r * inv_deg
        return cl0, cl1, cr0, cr1

    cf1 = coefs(1.0)
    cf10 = coefs(10.0)

    def pconv(xd, xu, cf, w_ref):
        cl0, cl1, cr0, cr1 = cf
        p = xd * cl0 + xu * cr0
        q = xd * cl1 + xu * cr1
        pq = jnp.concatenate([p, q], axis=1)
        return jnp.dot(pq, w_ref[:, :], preferred_element_type=jnp.float32)

    def gated(h, fpad, g_ref):
        gate = jax.nn.sigmoid(h[:, fpad - _LANE:])
        return h * jnp.dot(gate, g_ref[:, :],
                           preferred_element_type=jnp.float32)

    x = pconv(_sd(X), _su(X), cf1, w0_ref)

    for wa, wb, ws, gm, fpad in (
        (wa1_ref, wb1_ref, ws1_ref, g1_ref, 128),
        (wa2_ref, wb2_ref, ws2_ref, g2_ref, 256),
        (wa3_ref, wb3_ref, ws3_ref, g3_ref, 512),
        (wa4_ref, wb4_ref, ws4_ref, g4_ref, 384),
    ):
        xd, xu = _sd(x), _su(x)
        h = gated(pconv(xd, xu, cf1, wa), fpad, gm)
        h = gated(pconv(_sd(h), _su(h), cf1, wb), fpad, gm)
        x = pconv(xd, xu, cf10, ws) + h

    out = jnp.sqrt(jnp.dot(x * x, pool_ref[:, :],
                           preferred_element_type=jnp.float32) + 1e-12)
    out_ref[:, :] = out[_H:_H + block, :]


def _prep_conv_w(w, in_pad, out_pad):
    r, o, i = w.shape
    wt = jnp.transpose(w, (0, 2, 1))
    wt = jnp.pad(wt, ((0, 0), (0, in_pad - i), (0, out_pad - o)))
    return jnp.concatenate([wt[0], wt[1]], axis=0)


def _gate_expand(nf, fd, fpad):
    rows = jax.lax.broadcasted_iota(jnp.int32, (_LANE, fpad), 0)
    cols = jax.lax.broadcasted_iota(jnp.int32, (_LANE, fpad), 1)
    lane = rows + (fpad - _LANE)
    return ((lane == nf * fd + cols // fd) & (cols < nf * fd)).astype(
        jnp.float32)


def _pool_mat(nf, fd, in_pad):
    rows = jax.lax.broadcasted_iota(jnp.int32, (in_pad, _LANE), 0)
    cols = jax.lax.broadcasted_iota(jnp.int32, (in_pad, _LANE), 1)
    return ((rows // fd == cols) & (rows < nf * fd)).astype(jnp.float32)


@jax.jit
def kernel(input, W0, W1a, W1b, W1s, W2a, W2b, W2s, W3a, W3b, W3s, W4a,
           W4b, W4s):
    n = input.shape[0]
    block = 2000 if n >= 2000 else max(8, (n + 7) // 8 * 8)
    grid = -(-n // block)
    total = grid * block + 2 * _H
    tile = block + 2 * _H

    xp = jnp.zeros((total, _LANE), jnp.float32)
    xp = xp.at[_H:_H + n, :3].set(input)

    w0 = _prep_conv_w(W0, _LANE, _LANE)
    wa1 = _prep_conv_w(W1a, 128, 128)
    wb1 = _prep_conv_w(W1b, 128, 128)
    ws1 = _prep_conv_w(W1s, 128, 128)
    wa2 = _prep_conv_w(W2a, 128, 256)
    wb2 = _prep_conv_w(W2b, 256, 256)
    ws2 = _prep_conv_w(W2s, 128, 256)
    wa3 = _prep_conv_w(W3a, 256, 512)
    wb3 = _prep_conv_w(W3b, 512, 512)
    ws3 = _prep_conv_w(W3s, 256, 512)
    wa4 = _prep_conv_w(W4a, 512, 384)
    wb4 = _prep_conv_w(W4b, 384, 384)
    ws4 = _prep_conv_w(W4s, 512, 384)
    g1 = _gate_expand(2, 40, 128)
    g2 = _gate_expand(6, 40, 256)
    g3 = _gate_expand(12, 40, 512)
    g4 = _gate_expand(8, 40, 384)
    pool = _pool_mat(8, 40, 384)

    def full(a):
        return pl.BlockSpec(a.shape, lambda t: (0,) * a.ndim)

    operands = (xp, w0, wa1, wb1, ws1, g1, wa2, wb2, ws2, g2,
                wa3, wb3, ws3, g3, wa4, wb4, ws4, g4, pool)
    out = pl.pallas_call(
        functools.partial(_net_kernel, n_nodes=n, block=block, tile=tile),
        grid=(grid,),
        in_specs=[full(a) for a in operands],
        out_specs=pl.BlockSpec((block, _LANE), lambda t: (t, 0)),
        out_shape=jax.ShapeDtypeStruct((grid * block, _LANE), jnp.float32),
    )(*operands)
    return out[:n, :8]
